# baseline (speedup 1.0000x reference)
import functools

import numpy as np
import jax
import jax.numpy as jnp
from jax import lax
from jax.experimental import pallas as pl
from jax.experimental.pallas import tpu as pltpu
from jax.experimental.pallas import tpu_sc as plsc

N_EXP = 16
INNER = 4
BIGGER = 16
UNITS = N_EXP * INNER * BIGGER
D = 1024
BS = 512
NEG_INF = float("-inf")

_G = np.zeros((UNITS, N_EXP * INNER), np.float32)
_G[np.arange(UNITS), np.arange(UNITS) // BIGGER] = 1.0 / BIGGER


_H = UNITS // 2


def _tc_body(x_ref, wout_ref, g_ref, wg_hbm, wu_hbm, logits_ref, inner_ref,
             wg0, wu0, wg1, wu1, s0, s1, s2, s3):
    cp0 = pltpu.async_copy(wg_hbm.at[pl.ds(0, _H), :], wg0, s0)
    cp1 = pltpu.async_copy(wu_hbm.at[pl.ds(0, _H), :], wu0, s1)
    cp2 = pltpu.async_copy(wg_hbm.at[pl.ds(_H, _H), :], wg1, s2)
    cp3 = pltpu.async_copy(wu_hbm.at[pl.ds(_H, _H), :], wu1, s3)

    xb = x_ref[...].astype(jnp.bfloat16)
    wout = wout_ref[...].astype(jnp.bfloat16)
    nt = (((1,), (1,)), ((), ()))
    logits_ref[...] = lax.dot_general(
        xb, wout, nt, preferred_element_type=jnp.float32)

    def half(wg_buf, wu_buf, g_half):
        wg = wg_buf[...].astype(jnp.bfloat16)
        wu = wu_buf[...].astype(jnp.bfloat16)
        g = lax.dot_general(xb, wg, nt, preferred_element_type=jnp.float32)
        u = lax.dot_general(xb, wu, nt, preferred_element_type=jnp.float32)
        s = jnp.abs(u * (g * jax.nn.sigmoid(g)))
        return lax.dot_general(
            s, g_half, (((1,), (0,)), ((), ())),
            precision=lax.Precision.HIGHEST,
            preferred_element_type=jnp.float32)

    cp0.wait()
    cp1.wait()
    part0 = half(wg0, wu0, g_ref[0:_H, :])
    cp2.wait()
    cp3.wait()
    part1 = half(wg1, wu1, g_ref[_H:UNITS, :])
    inner_ref[...] = part0 + part1


def _tc_call(x, wout, wg2, wu2):
    return pl.pallas_call(
        _tc_body,
        in_specs=[
            pl.BlockSpec(memory_space=pltpu.VMEM),
            pl.BlockSpec(memory_space=pltpu.VMEM),
            pl.BlockSpec(memory_space=pltpu.VMEM),
            pl.BlockSpec(memory_space=pl.ANY),
            pl.BlockSpec(memory_space=pl.ANY),
        ],
        out_shape=[
            jax.ShapeDtypeStruct((BS, N_EXP), jnp.float32),
            jax.ShapeDtypeStruct((BS, N_EXP * INNER), jnp.float32),
        ],
        scratch_shapes=[
            pltpu.VMEM((_H, D), jnp.float32),
            pltpu.VMEM((_H, D), jnp.float32),
            pltpu.VMEM((_H, D), jnp.float32),
            pltpu.VMEM((_H, D), jnp.float32),
            pltpu.SemaphoreType.DMA,
            pltpu.SemaphoreType.DMA,
            pltpu.SemaphoreType.DMA,
            pltpu.SemaphoreType.DMA,
        ],
    )(x, wout, _G, wg2, wu2)


_SC_CORES = 1
_GROUPS = BS // (16 * 16 * _SC_CORES)


def _sc_process(lt_v, it_v, ov_v):
    lane = lax.iota(jnp.int32, 16)
    zeros = jnp.zeros((16,), jnp.int32)
    ninf = jnp.full((16,), NEG_INF, jnp.float32)

    def top2_step(c, carry):
        m0, e0, m1, e1 = carry
        l = plsc.load_gather(lt_v, [lane, zeros + c])
        gt0 = l > m0
        gt1 = l > m1
        e1 = jnp.where(gt0, e0, jnp.where(gt1, c, e1))
        m1 = jnp.where(gt0, m0, jnp.where(gt1, l, m1))
        e0 = jnp.where(gt0, c, e0)
        m0 = jnp.where(gt0, l, m0)
        return m0, e0, m1, e1

    l0 = plsc.load_gather(lt_v, [lane, zeros])
    m0, e0, m1, e1 = lax.fori_loop(
        1, N_EXP, top2_step, (l0, zeros, ninf, zeros))

    w0 = 1.0 / (1.0 + jnp.exp(m1 - m0))
    w1 = 1.0 - w0

    def inner_top2(e):
        base = e * INNER

        def step(i, carry):
            s0, i0, s1, i1 = carry
            s = plsc.load_gather(it_v, [lane, base + i])
            gt0 = s > s0
            gt1 = s > s1
            i1 = jnp.where(gt0, i0, jnp.where(gt1, i, i1))
            s1 = jnp.where(gt0, s0, jnp.where(gt1, s, s1))
            i0 = jnp.where(gt0, i, i0)
            s0 = jnp.where(gt0, s, s0)
            return s0, i0, s1, i1

        sA = plsc.load_gather(it_v, [lane, base])
        _, i0, _, i1 = lax.fori_loop(
            1, INNER, step, (sA, zeros, ninf, zeros))
        return base + i0, base + i1

    idA0, idA1 = inner_top2(e0)
    idB0, idB1 = inner_top2(e1)

    cols = [w0, w0, w1, w1,
            (idA0).astype(jnp.float32), (idA1).astype(jnp.float32),
            (idB0).astype(jnp.float32), (idB1).astype(jnp.float32)]
    for j, v in enumerate(cols):
        plsc.store_scatter(ov_v, [lane, zeros + j], v)


def _sc_body(logits_hbm, inner_hbm, out_hbm,
             lt0, it0, ov0, lt1, it1, ov1,
             sem_l0, sem_i0, sem_l1, sem_i1, sem_o):
    sid = lax.axis_index("s")
    tA = pl.multiple_of(sid * 32, 16)
    tB = pl.multiple_of(sid * 32 + 16, 16)
    cpl0 = pltpu.async_copy(logits_hbm.at[pl.ds(tA, 16), :], lt0, sem_l0)
    cpi0 = pltpu.async_copy(inner_hbm.at[pl.ds(tA, 16), :], it0, sem_i0)
    cpl1 = pltpu.async_copy(logits_hbm.at[pl.ds(tB, 16), :], lt1, sem_l1)
    cpi1 = pltpu.async_copy(inner_hbm.at[pl.ds(tB, 16), :], it1, sem_i1)
    cpl0.wait()
    cpi0.wait()
    _sc_process(lt0, it0, ov0)
    cpo0 = pltpu.async_copy(ov0, out_hbm.at[pl.ds(tA, 16), :], sem_o)
    cpl1.wait()
    cpi1.wait()
    _sc_process(lt1, it1, ov1)
    cpo1 = pltpu.async_copy(ov1, out_hbm.at[pl.ds(tB, 16), :], sem_o)
    cpo0.wait()
    cpo1.wait()


@functools.lru_cache(maxsize=1)
def _sc_call():
    return pl.kernel(
        _sc_body,
        out_type=jax.ShapeDtypeStruct((BS, 8), jnp.float32),
        mesh=plsc.VectorSubcoreMesh(
            core_axis_name="c", subcore_axis_name="s", num_cores=_SC_CORES),
        compiler_params=pltpu.CompilerParams(needs_layout_passes=False),
        scratch_types=[
            pltpu.VMEM((16, N_EXP), jnp.float32),
            pltpu.VMEM((16, N_EXP * INNER), jnp.float32),
            pltpu.VMEM((16, 8), jnp.float32),
            pltpu.VMEM((16, N_EXP), jnp.float32),
            pltpu.VMEM((16, N_EXP * INNER), jnp.float32),
            pltpu.VMEM((16, 8), jnp.float32),
            pltpu.SemaphoreType.DMA,
            pltpu.SemaphoreType.DMA,
            pltpu.SemaphoreType.DMA,
            pltpu.SemaphoreType.DMA,
            pltpu.SemaphoreType.DMA,
        ],
    )


def kernel(x, out_gate_weight, stacked_in_gate_weights, stacked_in_up_weights):
    wg2 = stacked_in_gate_weights.reshape(UNITS, D)
    wu2 = stacked_in_up_weights.reshape(UNITS, D)
    logits, inner = _tc_call(x, out_gate_weight, wg2, wu2)
    buf = _sc_call()(logits, inner)
    return buf[:, :4], buf[:, 4:].astype(jnp.int32)


# speedup vs baseline: 1.0085x; 1.0085x over previous
import functools

import numpy as np
import jax
import jax.numpy as jnp
from jax import lax
from jax.experimental import pallas as pl
from jax.experimental.pallas import tpu as pltpu
from jax.experimental.pallas import tpu_sc as plsc

N_EXP = 16
INNER = 4
BIGGER = 16
UNITS = N_EXP * INNER * BIGGER
D = 1024
BS = 512
NEG_INF = float("-inf")

_G = np.zeros((UNITS, N_EXP * INNER), np.float32)
_G[np.arange(UNITS), np.arange(UNITS) // BIGGER] = 1.0 / BIGGER


def _tc_body(x_ref, wout_ref, wg_ref, wu_ref, g_ref, logits_ref, inner_ref):
    xb = x_ref[...].astype(jnp.bfloat16)
    wout = wout_ref[...].astype(jnp.bfloat16)
    nt = (((1,), (1,)), ((), ()))
    logits_ref[...] = lax.dot_general(
        xb, wout, nt, preferred_element_type=jnp.float32)
    wg = wg_ref[...].astype(jnp.bfloat16)
    wu = wu_ref[...].astype(jnp.bfloat16)
    g = lax.dot_general(xb, wg, nt, preferred_element_type=jnp.float32)
    u = lax.dot_general(xb, wu, nt, preferred_element_type=jnp.float32)
    s = jnp.abs(u * (g * jax.nn.sigmoid(g)))
    inner_ref[...] = lax.dot_general(
        s, g_ref[...], (((1,), (0,)), ((), ())),
        precision=lax.Precision.HIGHEST,
        preferred_element_type=jnp.float32)


def _tc_call(x, wout, wg2, wu2):
    return pl.pallas_call(
        _tc_body,
        out_shape=[
            jax.ShapeDtypeStruct((BS, N_EXP), jnp.float32),
            jax.ShapeDtypeStruct((BS, N_EXP * INNER), jnp.float32),
        ],
    )(x, wout, wg2, wu2, _G)


_SC_CORES = 1
_GROUPS = BS // (16 * 16 * _SC_CORES)


def _sc_process(lt_v, it_v, ov_v):
    lane = lax.iota(jnp.int32, 16)
    zeros = jnp.zeros((16,), jnp.int32)
    ninf = jnp.full((16,), NEG_INF, jnp.float32)

    def top2_step(c, carry):
        m0, e0, m1, e1 = carry
        l = plsc.load_gather(lt_v, [lane, zeros + c])
        gt0 = l > m0
        gt1 = l > m1
        e1 = jnp.where(gt0, e0, jnp.where(gt1, c, e1))
        m1 = jnp.where(gt0, m0, jnp.where(gt1, l, m1))
        e0 = jnp.where(gt0, c, e0)
        m0 = jnp.where(gt0, l, m0)
        return m0, e0, m1, e1

    l0 = plsc.load_gather(lt_v, [lane, zeros])
    m0, e0, m1, e1 = lax.fori_loop(
        1, N_EXP, top2_step, (l0, zeros, ninf, zeros))

    w0 = 1.0 / (1.0 + jnp.exp(m1 - m0))
    w1 = 1.0 - w0

    def inner_top2(e):
        base = e * INNER

        def step(i, carry):
            s0, i0, s1, i1 = carry
            s = plsc.load_gather(it_v, [lane, base + i])
            gt0 = s > s0
            gt1 = s > s1
            i1 = jnp.where(gt0, i0, jnp.where(gt1, i, i1))
            s1 = jnp.where(gt0, s0, jnp.where(gt1, s, s1))
            i0 = jnp.where(gt0, i, i0)
            s0 = jnp.where(gt0, s, s0)
            return s0, i0, s1, i1

        sA = plsc.load_gather(it_v, [lane, base])
        _, i0, _, i1 = lax.fori_loop(
            1, INNER, step, (sA, zeros, ninf, zeros))
        return base + i0, base + i1

    idA0, idA1 = inner_top2(e0)
    idB0, idB1 = inner_top2(e1)

    cols = [w0, w0, w1, w1,
            (idA0).astype(jnp.float32), (idA1).astype(jnp.float32),
            (idB0).astype(jnp.float32), (idB1).astype(jnp.float32)]
    for j, v in enumerate(cols):
        plsc.store_scatter(ov_v, [lane, zeros + j], v)


def _sc_body(logits_hbm, inner_hbm, out_hbm,
             lt0, it0, ov0, lt1, it1, ov1,
             sem_l0, sem_i0, sem_l1, sem_i1, sem_o):
    sid = lax.axis_index("s")
    tA = pl.multiple_of(sid * 32, 16)
    tB = pl.multiple_of(sid * 32 + 16, 16)
    cpl0 = pltpu.async_copy(logits_hbm.at[pl.ds(tA, 16), :], lt0, sem_l0)
    cpi0 = pltpu.async_copy(inner_hbm.at[pl.ds(tA, 16), :], it0, sem_i0)
    cpl1 = pltpu.async_copy(logits_hbm.at[pl.ds(tB, 16), :], lt1, sem_l1)
    cpi1 = pltpu.async_copy(inner_hbm.at[pl.ds(tB, 16), :], it1, sem_i1)
    cpl0.wait()
    cpi0.wait()
    _sc_process(lt0, it0, ov0)
    cpo0 = pltpu.async_copy(ov0, out_hbm.at[pl.ds(tA, 16), :], sem_o)
    cpl1.wait()
    cpi1.wait()
    _sc_process(lt1, it1, ov1)
    cpo1 = pltpu.async_copy(ov1, out_hbm.at[pl.ds(tB, 16), :], sem_o)
    cpo0.wait()
    cpo1.wait()


@functools.lru_cache(maxsize=1)
def _sc_call():
    return pl.kernel(
        _sc_body,
        out_type=jax.ShapeDtypeStruct((BS, 8), jnp.float32),
        mesh=plsc.VectorSubcoreMesh(
            core_axis_name="c", subcore_axis_name="s", num_cores=_SC_CORES),
        compiler_params=pltpu.CompilerParams(needs_layout_passes=False),
        scratch_types=[
            pltpu.VMEM((16, N_EXP), jnp.float32),
            pltpu.VMEM((16, N_EXP * INNER), jnp.float32),
            pltpu.VMEM((16, 8), jnp.float32),
            pltpu.VMEM((16, N_EXP), jnp.float32),
            pltpu.VMEM((16, N_EXP * INNER), jnp.float32),
            pltpu.VMEM((16, 8), jnp.float32),
            pltpu.SemaphoreType.DMA,
            pltpu.SemaphoreType.DMA,
            pltpu.SemaphoreType.DMA,
            pltpu.SemaphoreType.DMA,
            pltpu.SemaphoreType.DMA,
        ],
    )


def kernel(x, out_gate_weight, stacked_in_gate_weights, stacked_in_up_weights):
    wg2 = stacked_in_gate_weights.reshape(UNITS, D)
    wu2 = stacked_in_up_weights.reshape(UNITS, D)
    logits, inner = _tc_call(x, out_gate_weight, wg2, wu2)
    buf = _sc_call()(logits, inner)
    return buf[:, :4], buf[:, 4:].astype(jnp.int32)
